# baseline (speedup 1.0000x reference)
import jax
import jax.numpy as jnp
from jax.experimental import pallas as pl
from jax.experimental.pallas import tpu as pltpu

_TOKEN_BLOCK = 512
_NBUF = 3


def _router_body(x_hbm, w_ref, b_ref, o_ref, buf, sems):
    tokens = o_ref.shape[0]
    blk = _TOKEN_BLOCK
    nsteps = tokens // blk

    def copy_in(step, slot):
        return pltpu.make_async_copy(
            x_hbm.at[pl.ds(step * blk, blk), :], buf.at[slot], sems.at[slot])

    for s in range(min(_NBUF, nsteps)):
        copy_in(s, s).start()

    for i in range(nsteps):
        slot = i % _NBUF
        copy_in(i, slot).wait()
        o_ref[pl.ds(i * blk, blk), :] = jax.lax.dot_general(
            buf[slot], w_ref[...],
            dimension_numbers=(((1,), (1,)), ((), ())),
            preferred_element_type=jnp.float32,
        ) + b_ref[...]
        nxt = i + _NBUF
        if nxt < nsteps:
            copy_in(nxt, slot).start()


def kernel(x, W, b):
    tokens, d = x.shape
    n_experts = W.shape[0]
    return pl.pallas_call(
        _router_body,
        in_specs=[
            pl.BlockSpec(memory_space=pltpu.MemorySpace.HBM),
            pl.BlockSpec(memory_space=pltpu.MemorySpace.VMEM),
            pl.BlockSpec(memory_space=pltpu.MemorySpace.VMEM),
        ],
        out_specs=pl.BlockSpec(memory_space=pltpu.MemorySpace.VMEM),
        out_shape=jax.ShapeDtypeStruct((tokens, n_experts), jnp.float32),
        scratch_shapes=[
            pltpu.MemorySpace.VMEM((_NBUF, _TOKEN_BLOCK, d), jnp.float32),
            pltpu.SemaphoreType.DMA((_NBUF,)),
        ],
    )(x, W, b.reshape(1, n_experts))
